# Initial kernel scaffold; baseline (speedup 1.0000x reference)
import jax
import jax.numpy as jnp
from jax.experimental import pallas as pl


def kernel(X, edge_index, edge_embedding, static_edge_features, W_out, b_out):
    raise NotImplementedError("write your pallas kernel here")


# speedup vs baseline: 3.7425x; 3.7425x over previous
import functools

import jax
import jax.numpy as jnp
from jax import lax
from jax.experimental import pallas as pl
from jax.experimental.pallas import tpu as pltpu
from jax.experimental.pallas import tpu_sc as plsc

NC, NS, L = 2, 16, 16
NW = NC * NS


def _node_scores_body(w2_ref, x_ref, out_ref):
    out_ref[...] = lax.dot_general(
        w2_ref[...], x_ref[...],
        dimension_numbers=(((1,), (1,)), ((), ())),
        preferred_element_type=jnp.float32,
    )


def _node_scores(w2, X):
    N = X.shape[0]
    return pl.pallas_call(
        _node_scores_body,
        out_shape=jax.ShapeDtypeStruct((2, N), jnp.float32),
    )(w2, X)


def _gsum_body(scores_hbm, ei_hbm, out_hbm, src_v, dst_v, ps_v, pd_v, out_v):
    epw = out_v.shape[0]
    n = ps_v.shape[0]
    e = out_hbm.shape[0]
    wid = lax.axis_index("s") * NC + lax.axis_index("c")
    base = wid * epw
    pltpu.sync_copy(scores_hbm.at[pl.ds(0, n)], ps_v)
    pltpu.sync_copy(scores_hbm.at[pl.ds(n, n)], pd_v)
    pltpu.sync_copy(ei_hbm.at[pl.ds(base, epw)], src_v)
    pltpu.sync_copy(ei_hbm.at[pl.ds(e + base, epw)], dst_v)

    def body(i, _):
        sl = pl.ds(i * L, L)
        g = plsc.load_gather(ps_v, [src_v[sl]])
        g = g + plsc.load_gather(pd_v, [dst_v[sl]])
        out_v[sl] = g
        return 0

    lax.fori_loop(0, epw // L, body, 0)
    pltpu.sync_copy(out_v, out_hbm.at[pl.ds(base, epw)])


def _gsum(scores, edge_index):
    N = scores.shape[1]
    E = edge_index.shape[1]
    epw = E // NW
    scores = scores.reshape(2 * N)
    edge_index = edge_index.reshape(2 * E)
    mesh = plsc.VectorSubcoreMesh(
        core_axis_name="c", subcore_axis_name="s", num_cores=NC, num_subcores=NS
    )
    return pl.kernel(
        _gsum_body,
        out_type=jax.ShapeDtypeStruct((E,), jnp.float32),
        mesh=mesh,
        compiler_params=pltpu.CompilerParams(needs_layout_passes=False),
        scratch_types=[
            pltpu.VMEM((epw,), jnp.int32),
            pltpu.VMEM((epw,), jnp.int32),
            pltpu.VMEM((N,), jnp.float32),
            pltpu.VMEM((N,), jnp.float32),
            pltpu.VMEM((epw,), jnp.float32),
        ],
    )(scores, edge_index)


def _edge_out_body(wemb_ref, wst_ref, b_ref, emb_ref, st_ref, g_ref, out_ref):
    be = out_ref.shape[2]
    et = lax.dot_general(
        wemb_ref[...], emb_ref[...],
        dimension_numbers=(((1,), (1,)), ((), ())),
        preferred_element_type=jnp.float32,
    )
    st = lax.dot_general(
        wst_ref[...], st_ref[...],
        dimension_numbers=(((1,), (1,)), ((), ())),
        preferred_element_type=jnp.float32,
    )
    out_ref[...] = (et + st + g_ref[0] + b_ref[0]).reshape(1, 1, be)


def _edge_out(w_emb, w_st, b, emb, static, gsum, be):
    E, EMB = emb.shape
    S = static.shape[1]
    nb = E // be
    g3 = gsum.reshape(nb, 1, be)
    out3 = pl.pallas_call(
        _edge_out_body,
        grid=(nb,),
        in_specs=[
            pl.BlockSpec((1, EMB), lambda i: (0, 0)),
            pl.BlockSpec((1, S), lambda i: (0, 0)),
            pl.BlockSpec(memory_space=pltpu.SMEM),
            pl.BlockSpec((be, EMB), lambda i: (i, 0)),
            pl.BlockSpec((be, S), lambda i: (i, 0)),
            pl.BlockSpec((1, 1, be), lambda i: (i, 0, 0)),
        ],
        out_specs=pl.BlockSpec((1, 1, be), lambda i: (i, 0, 0)),
        out_shape=jax.ShapeDtypeStruct((nb, 1, be), jnp.float32),
    )(w_emb, w_st, b, emb, static, g3)
    return out3.reshape(E, 1)


def kernel(X, edge_index, edge_embedding, static_edge_features, W_out, b_out):
    N, H = X.shape
    E = edge_index.shape[1]
    EMB = edge_embedding.shape[1]
    w2 = W_out[:, : 2 * H].reshape(2, H)
    w_emb = W_out[:, 2 * H : 2 * H + EMB]
    w_st = W_out[:, 2 * H + EMB :]
    scores = _node_scores(w2, X)
    gsum = _gsum(scores, edge_index)
    return _edge_out(w_emb, w_st, b_out, edge_embedding, static_edge_features,
                     gsum, be=6400)


# speedup vs baseline: 14.6739x; 3.9209x over previous
import functools

import jax
import jax.numpy as jnp
from jax import lax
from jax.experimental import pallas as pl
from jax.experimental.pallas import tpu as pltpu
from jax.experimental.pallas import tpu_sc as plsc

NC, NS, L = 2, 16, 16
NW = NC * NS


def _node_scores_body(w2_ref, x_ref, out_ref):
    out_ref[...] = lax.dot_general(
        w2_ref[...], x_ref[...],
        dimension_numbers=(((1,), (1,)), ((), ())),
        preferred_element_type=jnp.float32,
    )


def _node_scores(w2, X):
    N = X.shape[0]
    return pl.pallas_call(
        _node_scores_body,
        out_shape=jax.ShapeDtypeStruct((2, N), jnp.float32),
    )(w2, X)


def _gsum_body(scores_hbm, ei_hbm, out_hbm, src_v, dst_v, ps_v, pd_v, out_v):
    epw = out_v.shape[0]
    n = ps_v.shape[0]
    e = out_hbm.shape[0]
    wid = lax.axis_index("s") * NC + lax.axis_index("c")
    base = wid * epw
    pltpu.sync_copy(scores_hbm.at[pl.ds(0, n)], ps_v)
    pltpu.sync_copy(scores_hbm.at[pl.ds(n, n)], pd_v)
    pltpu.sync_copy(ei_hbm.at[pl.ds(base, epw)], src_v)
    pltpu.sync_copy(ei_hbm.at[pl.ds(e + base, epw)], dst_v)

    def body(i, _):
        sl = pl.ds(i * L, L)
        g = plsc.load_gather(ps_v, [src_v[sl]])
        g = g + plsc.load_gather(pd_v, [dst_v[sl]])
        out_v[sl] = g
        return 0

    lax.fori_loop(0, epw // L, body, 0)
    pltpu.sync_copy(out_v, out_hbm.at[pl.ds(base, epw)])


def _gsum(scores, edge_index):
    N = scores.shape[1]
    E = edge_index.shape[1]
    epw = E // NW
    scores = scores.reshape(2 * N)
    edge_index = edge_index.reshape(2 * E)
    mesh = plsc.VectorSubcoreMesh(
        core_axis_name="c", subcore_axis_name="s", num_cores=NC, num_subcores=NS
    )
    return pl.kernel(
        _gsum_body,
        out_type=jax.ShapeDtypeStruct((E,), jnp.float32),
        mesh=mesh,
        compiler_params=pltpu.CompilerParams(needs_layout_passes=False),
        scratch_types=[
            pltpu.VMEM((epw,), jnp.int32),
            pltpu.VMEM((epw,), jnp.int32),
            pltpu.VMEM((N,), jnp.float32),
            pltpu.VMEM((N,), jnp.float32),
            pltpu.VMEM((epw,), jnp.float32),
        ],
    )(scores, edge_index)


def _edge_out_body(wemb_ref, wst_ref, b_ref, emb_ref, st_ref, g_ref, out_ref):
    et = lax.dot_general(
        wemb_ref[...], emb_ref[...],
        dimension_numbers=(((1,), (0,)), ((), ())),
        preferred_element_type=jnp.float32,
    )
    st = lax.dot_general(
        wst_ref[...], st_ref[...],
        dimension_numbers=(((1,), (0,)), ((), ())),
        preferred_element_type=jnp.float32,
    )
    out_ref[...] = et + st + g_ref[...] + b_ref[0]


def _edge_out(w_emb, w_st, b, emb, static, gsum, be):
    E, EMB = emb.shape
    S = static.shape[1]
    nb = E // be
    emb_t = emb.T
    st_t = static.T
    g2 = gsum.reshape(1, E)
    out2 = pl.pallas_call(
        _edge_out_body,
        grid=(nb,),
        in_specs=[
            pl.BlockSpec((1, EMB), lambda i: (0, 0)),
            pl.BlockSpec((1, S), lambda i: (0, 0)),
            pl.BlockSpec(memory_space=pltpu.SMEM),
            pl.BlockSpec((EMB, be), lambda i: (0, i)),
            pl.BlockSpec((S, be), lambda i: (0, i)),
            pl.BlockSpec((1, be), lambda i: (0, i)),
        ],
        out_specs=pl.BlockSpec((1, be), lambda i: (0, i)),
        out_shape=jax.ShapeDtypeStruct((1, E), jnp.float32),
    )(w_emb, w_st, b, emb_t, st_t, g2)
    return out2.reshape(E, 1)


def kernel(X, edge_index, edge_embedding, static_edge_features, W_out, b_out):
    N, H = X.shape
    E = edge_index.shape[1]
    EMB = edge_embedding.shape[1]
    w2 = W_out[:, : 2 * H].reshape(2, H)
    w_emb = W_out[:, 2 * H : 2 * H + EMB]
    w_st = W_out[:, 2 * H + EMB :]
    scores = _node_scores(w2, X)
    gsum = _gsum(scores, edge_index)
    return _edge_out(w_emb, w_st, b_out, edge_embedding, static_edge_features,
                     gsum, be=6400)
